# Initial kernel scaffold; baseline (speedup 1.0000x reference)
import jax
import jax.numpy as jnp
from jax.experimental import pallas as pl


def kernel(x, edge_index, W1, a_src1, a_dst1, b1, W2, a_src2, a_dst2, b2):
    raise NotImplementedError("write your pallas kernel here")


# speedup vs baseline: 38.1410x; 38.1410x over previous
import functools

import jax
import jax.numpy as jnp
from jax import lax
from jax.experimental import pallas as pl
from jax.experimental.pallas import tpu as pltpu
from jax.experimental.pallas import tpu_sc as plsc

N = 10000
E = 320000
NP = 10240
HEADS1 = 8
MSG1 = 128
ROW1 = 144
MSG2 = 64
ROW2 = 80
NW = 32
EPW = E // NW
K = 80
NCH = EPW // K
RPT = NP // 16
RCH = 80
NRC = RPT // RCH


def _mm_kernel(x_ref, w_ref, o_ref):
    o_ref[...] = jnp.dot(x_ref[...], w_ref[...],
                         preferred_element_type=jnp.float32)


def _matmul(x, w, bm=256):
    m, k = x.shape
    n = w.shape[1]
    return pl.pallas_call(
        _mm_kernel,
        grid=(m // bm,),
        in_specs=[pl.BlockSpec((bm, k), lambda i: (i, 0)),
                  pl.BlockSpec((k, n), lambda i: (0, 0))],
        out_specs=pl.BlockSpec((bm, n), lambda i: (i, 0)),
        out_shape=jax.ShapeDtypeStruct((m, n), jnp.float32),
    )(x, w)


def _fin1_kernel(a0_ref, a1_ref, r_ref, b_ref, w_ref, o_ref):
    acc = a0_ref[...] + a1_ref[...]
    msg = acc[:, :MSG1]
    s = acc[:, MSG1:MSG1 + HEADS1]
    s_exp = jnp.dot(s, r_ref[...], preferred_element_type=jnp.float32)
    h = msg / (s_exp + 1e-16) + b_ref[...]
    h = jnp.where(h > 0, h, jnp.exp(h) - 1.0)
    o_ref[...] = jnp.dot(h, w_ref[...], preferred_element_type=jnp.float32)


def _finalize1_matmul2(a0, a1, rmat, b1, wbig2, bm=256):
    return pl.pallas_call(
        _fin1_kernel,
        grid=(NP // bm,),
        in_specs=[pl.BlockSpec((bm, ROW1), lambda i: (i, 0)),
                  pl.BlockSpec((bm, ROW1), lambda i: (i, 0)),
                  pl.BlockSpec((HEADS1, MSG1), lambda i: (0, 0)),
                  pl.BlockSpec((1, MSG1), lambda i: (0, 0)),
                  pl.BlockSpec((MSG1, 96), lambda i: (0, 0))],
        out_specs=pl.BlockSpec((bm, 96), lambda i: (i, 0)),
        out_shape=jax.ShapeDtypeStruct((NP, 96), jnp.float32),
    )(a0, a1, rmat, b1, wbig2)


def _fin2_kernel(a0_ref, a1_ref, b_ref, o_ref):
    acc = a0_ref[...] + a1_ref[...]
    msg = acc[:, :MSG2]
    s = acc[:, MSG2:MSG2 + 1]
    o_ref[...] = msg / (s + 1e-16) + b_ref[...]


def _finalize2(a0, a1, b2, bm=256):
    return pl.pallas_call(
        _fin2_kernel,
        grid=(NP // bm,),
        in_specs=[pl.BlockSpec((bm, ROW2), lambda i: (i, 0)),
                  pl.BlockSpec((bm, ROW2), lambda i: (i, 0)),
                  pl.BlockSpec((1, MSG2), lambda i: (0, 0))],
        out_specs=pl.BlockSpec((bm, MSG2), lambda i: (i, 0)),
        out_shape=jax.ShapeDtypeStruct((NP, MSG2), jnp.float32),
    )(a0, a1, b2)


def _edge_pass(row_w, msg_w, heads, srctab, dsttab, src_idx, dst_idx, zrows):
    cw = msg_w // heads
    mesh = plsc.VectorSubcoreMesh(core_axis_name="c", subcore_axis_name="s")

    @functools.partial(
        pl.kernel,
        mesh=mesh,
        compiler_params=pltpu.CompilerParams(use_tc_tiling_on_sc=False),
        out_type=jax.ShapeDtypeStruct((2, NP, row_w), jnp.float32),
        scratch_types=[
            pltpu.VMEM((K,), jnp.int32),
            pltpu.VMEM((K,), jnp.int32),
            pltpu.VMEM((K, row_w), jnp.float32),
            pltpu.VMEM((K, 16), jnp.float32),
            pltpu.VMEM_SHARED((NP, row_w), jnp.float32),
            pltpu.SemaphoreType.DMA,
        ],
    )
    def edge_kernel(srctab_hbm, dsttab_hbm, sidx_hbm, didx_hbm, z_hbm,
                    out_hbm, sidx, didx, rbuf, dbuf, acc, sem):
        cid = lax.axis_index("c")
        sid = lax.axis_index("s")
        wid = cid * 16 + sid

        pltpu.sync_copy(z_hbm, rbuf)
        def zbody(j, carry):
            pltpu.sync_copy(rbuf, acc.at[pl.ds(sid * RPT + j * RCH, RCH)])
            return carry
        lax.fori_loop(0, NRC, zbody, 0)
        plsc.subcore_barrier()

        ebase = wid * EPW

        def chunk(c, carry):
            eb = ebase + c * K
            pltpu.sync_copy(sidx_hbm.at[pl.ds(eb, K)], sidx)
            pltpu.sync_copy(didx_hbm.at[pl.ds(eb, K)], didx)
            pltpu.async_copy(srctab_hbm.at[sidx], rbuf, sem).wait()
            pltpu.async_copy(dsttab_hbm.at[didx], dbuf, sem).wait()

            def ebody(i, icarry):
                ev = rbuf[i, pl.ds(msg_w, 16)]
                dv = dbuf[i, pl.ds(0, 16)]
                e = ev + dv
                e = jnp.where(e >= 0, e, 0.2 * e)
                p = jnp.exp(e)
                rbuf[i, pl.ds(msg_w, 16)] = p
                for hd in range(heads):
                    ph = p[hd]
                    for q in range(cw // 16):
                        sl = hd * cw + q * 16
                        rbuf[i, pl.ds(sl, 16)] = rbuf[i, pl.ds(sl, 16)] * ph
                return icarry

            lax.fori_loop(0, K, ebody, 0)
            pltpu.sync_copy(rbuf, acc.at[didx], add=True)
            return carry

        lax.fori_loop(0, NCH, chunk, 0)
        plsc.subcore_barrier()

        def rbody(j, carry):
            r0 = sid * RPT + j * RCH
            pltpu.sync_copy(acc.at[pl.ds(r0, RCH)], rbuf)
            pltpu.sync_copy(rbuf, out_hbm.at[cid, pl.ds(r0, RCH)])
            return carry
        lax.fori_loop(0, NRC, rbody, 0)

    return edge_kernel(srctab, dsttab, src_idx, dst_idx, zrows)


@jax.jit
def kernel(x, edge_index, W1, a_src1, a_dst1, b1, W2, a_src2, a_dst2, b2):
    edge_index = edge_index.astype(jnp.int32)
    src = edge_index[0]
    dst = edge_index[1]

    eye8 = jnp.eye(HEADS1, dtype=jnp.float32)
    ms1 = (eye8[:, None, :] * a_src1[:, :, None]).reshape(MSG1, HEADS1)
    md1 = (eye8[:, None, :] * a_dst1[:, :, None]).reshape(MSG1, HEADS1)
    z8 = jnp.zeros((x.shape[1], HEADS1), jnp.float32)
    wbig1 = jnp.concatenate([W1, W1 @ ms1, z8, W1 @ md1, z8], axis=1)

    z15 = jnp.zeros((MSG1, 15), jnp.float32)
    wbig2 = jnp.concatenate(
        [W2, (W2 @ a_src2[0])[:, None], z15, (W2 @ a_dst2[0])[:, None], z15],
        axis=1)
    rmat = jnp.repeat(eye8, 16, axis=1)

    xp = jnp.pad(x, ((0, NP - N), (0, 0)))

    t1 = _matmul(xp, wbig1)
    srctab1 = t1[:, :ROW1]
    dsttab1 = t1[:, ROW1:160]
    z1 = jnp.zeros((K, ROW1), jnp.float32)
    accp1 = _edge_pass(ROW1, MSG1, HEADS1, srctab1, dsttab1, src, dst, z1)

    t2 = _finalize1_matmul2(accp1[0], accp1[1], rmat,
                            b1.reshape(1, MSG1), wbig2)
    srctab2 = t2[:, :ROW2]
    dsttab2 = t2[:, ROW2:96]
    z2 = jnp.zeros((K, ROW2), jnp.float32)
    accp2 = _edge_pass(ROW2, MSG2, 1, srctab2, dsttab2, src, dst, z2)

    out = _finalize2(accp2[0], accp2[1], b2.reshape(1, MSG2))
    return out[:N]


# speedup vs baseline: 69.8080x; 1.8303x over previous
import functools

import jax
import jax.numpy as jnp
from jax import lax
from jax.experimental import pallas as pl
from jax.experimental.pallas import tpu as pltpu
from jax.experimental.pallas import tpu_sc as plsc

N = 10000
E = 320000
NP = 10240
HEADS1 = 8
MSG1 = 128
ROW1 = 144
MSG2 = 64
ROW2 = 80
NW = 32
EPW = E // NW
K = 40
NCH = EPW // K
RPT = NP // 16
RCH = 40
NRC = RPT // RCH


def _mm_kernel(x_ref, w_ref, o_ref):
    o_ref[...] = jnp.dot(x_ref[...], w_ref[...],
                         preferred_element_type=jnp.float32)


def _matmul(x, w, bm=256):
    m, k = x.shape
    n = w.shape[1]
    return pl.pallas_call(
        _mm_kernel,
        grid=(m // bm,),
        in_specs=[pl.BlockSpec((bm, k), lambda i: (i, 0)),
                  pl.BlockSpec((k, n), lambda i: (0, 0))],
        out_specs=pl.BlockSpec((bm, n), lambda i: (i, 0)),
        out_shape=jax.ShapeDtypeStruct((m, n), jnp.float32),
    )(x, w)


def _fin1_kernel(a0_ref, a1_ref, r_ref, b_ref, w_ref, o_ref):
    acc = a0_ref[...] + a1_ref[...]
    msg = acc[:, :MSG1]
    s = acc[:, MSG1:MSG1 + HEADS1]
    s_exp = jnp.dot(s, r_ref[...], preferred_element_type=jnp.float32)
    h = msg / (s_exp + 1e-16) + b_ref[...]
    h = jnp.where(h > 0, h, jnp.exp(h) - 1.0)
    o_ref[...] = jnp.dot(h, w_ref[...], preferred_element_type=jnp.float32)


def _finalize1_matmul2(a0, a1, rmat, b1, wbig2, bm=256):
    return pl.pallas_call(
        _fin1_kernel,
        grid=(NP // bm,),
        in_specs=[pl.BlockSpec((bm, ROW1), lambda i: (i, 0)),
                  pl.BlockSpec((bm, ROW1), lambda i: (i, 0)),
                  pl.BlockSpec((HEADS1, MSG1), lambda i: (0, 0)),
                  pl.BlockSpec((1, MSG1), lambda i: (0, 0)),
                  pl.BlockSpec((MSG1, 96), lambda i: (0, 0))],
        out_specs=pl.BlockSpec((bm, 96), lambda i: (i, 0)),
        out_shape=jax.ShapeDtypeStruct((NP, 96), jnp.float32),
    )(a0, a1, rmat, b1, wbig2)


def _fin2_kernel(a0_ref, a1_ref, b_ref, o_ref):
    acc = a0_ref[...] + a1_ref[...]
    msg = acc[:, :MSG2]
    s = acc[:, MSG2:MSG2 + 1]
    o_ref[...] = msg / (s + 1e-16) + b_ref[...]


def _finalize2(a0, a1, b2, bm=256):
    return pl.pallas_call(
        _fin2_kernel,
        grid=(NP // bm,),
        in_specs=[pl.BlockSpec((bm, ROW2), lambda i: (i, 0)),
                  pl.BlockSpec((bm, ROW2), lambda i: (i, 0)),
                  pl.BlockSpec((1, MSG2), lambda i: (0, 0))],
        out_specs=pl.BlockSpec((bm, MSG2), lambda i: (i, 0)),
        out_shape=jax.ShapeDtypeStruct((NP, MSG2), jnp.float32),
    )(a0, a1, b2)


def _edge_pass(row_w, msg_w, heads, srctab, dsttab, src_idx, dst_idx, zrows):
    cw = msg_w // heads
    mesh = plsc.VectorSubcoreMesh(core_axis_name="c", subcore_axis_name="s")

    @functools.partial(
        pl.kernel,
        mesh=mesh,
        compiler_params=pltpu.CompilerParams(use_tc_tiling_on_sc=False),
        out_type=jax.ShapeDtypeStruct((2, NP, row_w), jnp.float32),
        scratch_types=[
            pltpu.VMEM((NCH, K), jnp.int32),
            pltpu.VMEM((NCH, K), jnp.int32),
            pltpu.VMEM((2, K, row_w), jnp.float32),
            pltpu.VMEM((2, K, 16), jnp.float32),
            pltpu.VMEM_SHARED((NP, row_w), jnp.float32),
            pltpu.SemaphoreType.DMA,
            pltpu.SemaphoreType.DMA,
        ],
    )
    def edge_kernel(srctab_hbm, dsttab_hbm, sidx_hbm, didx_hbm, z_hbm,
                    out_hbm, sidx, didx, rbuf, dbuf, acc, sg0, sg1):
        cid = lax.axis_index("c")
        sid = lax.axis_index("s")
        wid = cid * 16 + sid
        sg = (sg0, sg1)

        pltpu.sync_copy(z_hbm, rbuf.at[0])
        def zbody(j, carry):
            pltpu.sync_copy(rbuf.at[0],
                            acc.at[pl.ds(sid * RPT + j * RCH, RCH)])
            return carry
        lax.fori_loop(0, NRC, zbody, 0)
        plsc.subcore_barrier()

        pltpu.sync_copy(sidx_hbm.at[wid], sidx)
        pltpu.sync_copy(didx_hbm.at[wid], didx)

        def issue_gather(c, b):
            pltpu.async_copy(srctab_hbm.at[sidx.at[c]], rbuf.at[b], sg[b])
            pltpu.async_copy(dsttab_hbm.at[didx.at[c]], dbuf.at[b], sg[b])

        def wait_gather(c, b):
            pltpu.make_async_copy(srctab_hbm.at[sidx.at[c]], rbuf.at[b],
                                  sg[b]).wait()
            pltpu.make_async_copy(dsttab_hbm.at[didx.at[c]], dbuf.at[b],
                                  sg[b]).wait()

        def compute_scatter(c, b):
            def ebody(i, icarry):
                ev = rbuf[b, i, pl.ds(msg_w, 16)]
                dv = dbuf[b, i, pl.ds(0, 16)]
                e = ev + dv
                e = jnp.where(e >= 0, e, 0.2 * e)
                p = jnp.exp(e)
                rbuf[b, i, pl.ds(msg_w, 16)] = p
                for hd in range(heads):
                    ph = p[hd]
                    for q in range(cw // 16):
                        sl = hd * cw + q * 16
                        rbuf[b, i, pl.ds(sl, 16)] = (
                            rbuf[b, i, pl.ds(sl, 16)] * ph)
                return icarry
            lax.fori_loop(0, K, ebody, 0)
            pltpu.sync_copy(rbuf.at[b], acc.at[didx.at[c]], add=True)

        issue_gather(0, 0)

        def outer(t, carry):
            c0 = t * 2
            for b in range(2):
                c = c0 + b
                wait_gather(c, b)
                issue_gather(c + 1, 1 - b)
                compute_scatter(c, b)
            return carry

        lax.fori_loop(0, NCH // 2 - 1, outer, 0)
        wait_gather(NCH - 2, 0)
        issue_gather(NCH - 1, 1)
        compute_scatter(NCH - 2, 0)
        wait_gather(NCH - 1, 1)
        compute_scatter(NCH - 1, 1)

        plsc.subcore_barrier()

        def rbody(j, carry):
            r0 = sid * RPT + j * RCH
            pltpu.sync_copy(acc.at[pl.ds(r0, RCH)], rbuf.at[0])
            pltpu.sync_copy(rbuf.at[0], out_hbm.at[cid, pl.ds(r0, RCH)])
            return carry
        lax.fori_loop(0, NRC, rbody, 0)

    return edge_kernel(srctab, dsttab, src_idx.reshape(NW, NCH, K),
                       dst_idx.reshape(NW, NCH, K), zrows)


@jax.jit
def kernel(x, edge_index, W1, a_src1, a_dst1, b1, W2, a_src2, a_dst2, b2):
    edge_index = edge_index.astype(jnp.int32)
    src = edge_index[0]
    dst = edge_index[1]

    eye8 = jnp.eye(HEADS1, dtype=jnp.float32)
    ms1 = (eye8[:, None, :] * a_src1[:, :, None]).reshape(MSG1, HEADS1)
    md1 = (eye8[:, None, :] * a_dst1[:, :, None]).reshape(MSG1, HEADS1)
    z8 = jnp.zeros((x.shape[1], HEADS1), jnp.float32)
    wbig1 = jnp.concatenate([W1, W1 @ ms1, z8, W1 @ md1, z8], axis=1)

    z15 = jnp.zeros((MSG1, 15), jnp.float32)
    wbig2 = jnp.concatenate(
        [W2, (W2 @ a_src2[0])[:, None], z15, (W2 @ a_dst2[0])[:, None], z15],
        axis=1)
    rmat = jnp.repeat(eye8, 16, axis=1)

    xp = jnp.pad(x, ((0, NP - N), (0, 0)))

    t1 = _matmul(xp, wbig1)
    srctab1 = t1[:, :ROW1]
    dsttab1 = t1[:, ROW1:160]
    z1 = jnp.zeros((K, ROW1), jnp.float32)
    accp1 = _edge_pass(ROW1, MSG1, HEADS1, srctab1, dsttab1, src, dst, z1)

    t2 = _finalize1_matmul2(accp1[0], accp1[1], rmat,
                            b1.reshape(1, MSG1), wbig2)
    srctab2 = t2[:, :ROW2]
    dsttab2 = t2[:, ROW2:96]
    z2 = jnp.zeros((K, ROW2), jnp.float32)
    accp2 = _edge_pass(ROW2, MSG2, 1, srctab2, dsttab2, src, dst, z2)

    out = _finalize2(accp2[0], accp2[1], b2.reshape(1, MSG2))
    return out[:N]


# speedup vs baseline: 75.2215x; 1.0775x over previous
import functools

import jax
import jax.numpy as jnp
from jax import lax
from jax.experimental import pallas as pl
from jax.experimental.pallas import tpu as pltpu
from jax.experimental.pallas import tpu_sc as plsc

N = 10000
E = 320000
NP = 10240
HEADS1 = 8
MSG1 = 128
ROW1 = 144
MSG2 = 64
ROW2 = 80
NW = 32
EPW = E // NW
K = 40
NCH = EPW // K
RPT = NP // 16
RCH = 40
NRC = RPT // RCH


def _mm_kernel(x_ref, w_ref, o1_ref, o2_ref):
    t = jnp.dot(x_ref[...], w_ref[...], preferred_element_type=jnp.float32)
    o1_ref[...] = t[:, :ROW1]
    o2_ref[...] = t[:, ROW1:160]


def _tables1(x, w, bm=256):
    m = x.shape[0]
    k = x.shape[1]
    return pl.pallas_call(
        _mm_kernel,
        grid=(m // bm,),
        in_specs=[pl.BlockSpec((bm, k), lambda i: (i, 0)),
                  pl.BlockSpec((k, 160), lambda i: (0, 0))],
        out_specs=[pl.BlockSpec((bm, ROW1), lambda i: (i, 0)),
                   pl.BlockSpec((bm, 16), lambda i: (i, 0))],
        out_shape=[jax.ShapeDtypeStruct((m, ROW1), jnp.float32),
                   jax.ShapeDtypeStruct((m, 16), jnp.float32)],
    )(x, w)


def _fin1_kernel(a0_ref, a1_ref, r_ref, b_ref, w_ref, o1_ref, o2_ref):
    acc = a0_ref[...] + a1_ref[...]
    msg = acc[:, :MSG1]
    s = acc[:, MSG1:MSG1 + HEADS1]
    s_exp = jnp.dot(s, r_ref[...], preferred_element_type=jnp.float32)
    h = msg / (s_exp + 1e-16) + b_ref[...]
    h = jnp.where(h > 0, h, jnp.exp(h) - 1.0)
    t = jnp.dot(h, w_ref[...], preferred_element_type=jnp.float32)
    o1_ref[...] = t[:, :ROW2]
    o2_ref[...] = t[:, ROW2:96]


def _finalize1_matmul2(a0, a1, rmat, b1, wbig2, bm=256):
    return pl.pallas_call(
        _fin1_kernel,
        grid=(NP // bm,),
        in_specs=[pl.BlockSpec((bm, ROW1), lambda i: (i, 0)),
                  pl.BlockSpec((bm, ROW1), lambda i: (i, 0)),
                  pl.BlockSpec((HEADS1, MSG1), lambda i: (0, 0)),
                  pl.BlockSpec((1, MSG1), lambda i: (0, 0)),
                  pl.BlockSpec((MSG1, 96), lambda i: (0, 0))],
        out_specs=[pl.BlockSpec((bm, ROW2), lambda i: (i, 0)),
                   pl.BlockSpec((bm, 16), lambda i: (i, 0))],
        out_shape=[jax.ShapeDtypeStruct((NP, ROW2), jnp.float32),
                   jax.ShapeDtypeStruct((NP, 16), jnp.float32)],
    )(a0, a1, rmat, b1, wbig2)


def _fin2_kernel(a0_ref, a1_ref, b_ref, o_ref):
    acc = a0_ref[...] + a1_ref[...]
    msg = acc[:, :MSG2]
    s = acc[:, MSG2:MSG2 + 1]
    o_ref[...] = msg / (s + 1e-16) + b_ref[...]


def _finalize2(a0, a1, b2, bm=256):
    return pl.pallas_call(
        _fin2_kernel,
        grid=(NP // bm,),
        in_specs=[pl.BlockSpec((bm, ROW2), lambda i: (i, 0)),
                  pl.BlockSpec((bm, ROW2), lambda i: (i, 0)),
                  pl.BlockSpec((1, MSG2), lambda i: (0, 0))],
        out_specs=pl.BlockSpec((bm, MSG2), lambda i: (i, 0)),
        out_shape=jax.ShapeDtypeStruct((NP, MSG2), jnp.float32),
    )(a0, a1, b2)


def _edge_pass(row_w, msg_w, heads, srctab, dsttab, src_idx, dst_idx, zrows):
    cw = msg_w // heads
    mesh = plsc.VectorSubcoreMesh(core_axis_name="c", subcore_axis_name="s")

    @functools.partial(
        pl.kernel,
        mesh=mesh,
        compiler_params=pltpu.CompilerParams(use_tc_tiling_on_sc=False),
        out_type=jax.ShapeDtypeStruct((2, NP, row_w), jnp.float32),
        scratch_types=[
            pltpu.VMEM((NCH, K), jnp.int32),
            pltpu.VMEM((NCH, K), jnp.int32),
            pltpu.VMEM((2, K, row_w), jnp.float32),
            pltpu.VMEM((2, K, 16), jnp.float32),
            pltpu.VMEM_SHARED((NP, row_w), jnp.float32),
            pltpu.SemaphoreType.DMA,
            pltpu.SemaphoreType.DMA,
        ],
    )
    def edge_kernel(srctab_hbm, dsttab_hbm, sidx_hbm, didx_hbm, z_hbm,
                    out_hbm, sidx, didx, rbuf, dbuf, acc, sg0, sg1):
        cid = lax.axis_index("c")
        sid = lax.axis_index("s")
        wid = cid * 16 + sid
        sg = (sg0, sg1)

        pltpu.sync_copy(z_hbm, rbuf.at[0])
        def zbody(j, carry):
            pltpu.sync_copy(rbuf.at[0],
                            acc.at[pl.ds(sid * RPT + j * RCH, RCH)])
            return carry
        lax.fori_loop(0, NRC, zbody, 0)
        plsc.subcore_barrier()

        pltpu.sync_copy(sidx_hbm.at[wid], sidx)
        pltpu.sync_copy(didx_hbm.at[wid], didx)

        def issue_gather(c, b):
            pltpu.async_copy(srctab_hbm.at[sidx.at[c]], rbuf.at[b], sg[b])
            pltpu.async_copy(dsttab_hbm.at[didx.at[c]], dbuf.at[b], sg[b])

        def wait_gather(c, b):
            pltpu.make_async_copy(srctab_hbm.at[sidx.at[c]], rbuf.at[b],
                                  sg[b]).wait()
            pltpu.make_async_copy(dsttab_hbm.at[didx.at[c]], dbuf.at[b],
                                  sg[b]).wait()

        def compute_scatter(c, b):
            @plsc.parallel_loop(0, K, unroll=2)
            def ebody(i):
                ev = rbuf[b, i, pl.ds(msg_w, 16)]
                dv = dbuf[b, i, pl.ds(0, 16)]
                e = ev + dv
                e = jnp.where(e >= 0, e, 0.2 * e)
                p = jnp.exp(e)
                rbuf[b, i, pl.ds(msg_w, 16)] = p
                for hd in range(heads):
                    ph = p[hd]
                    for q in range(cw // 16):
                        sl = hd * cw + q * 16
                        rbuf[b, i, pl.ds(sl, 16)] = (
                            rbuf[b, i, pl.ds(sl, 16)] * ph)
            pltpu.sync_copy(rbuf.at[b], acc.at[didx.at[c]], add=True)

        issue_gather(0, 0)

        def outer(t, carry):
            c0 = t * 2
            for b in range(2):
                c = c0 + b
                wait_gather(c, b)
                issue_gather(c + 1, 1 - b)
                compute_scatter(c, b)
            return carry

        lax.fori_loop(0, NCH // 2 - 1, outer, 0)
        wait_gather(NCH - 2, 0)
        issue_gather(NCH - 1, 1)
        compute_scatter(NCH - 2, 0)
        wait_gather(NCH - 1, 1)
        compute_scatter(NCH - 1, 1)

        plsc.subcore_barrier()

        def rbody(j, carry):
            r0 = sid * RPT + j * RCH
            pltpu.sync_copy(acc.at[pl.ds(r0, RCH)], rbuf.at[0])
            pltpu.sync_copy(rbuf.at[0], out_hbm.at[cid, pl.ds(r0, RCH)])
            return carry
        lax.fori_loop(0, NRC, rbody, 0)

    return edge_kernel(srctab, dsttab, src_idx.reshape(NW, NCH, K),
                       dst_idx.reshape(NW, NCH, K), zrows)


@jax.jit
def kernel(x, edge_index, W1, a_src1, a_dst1, b1, W2, a_src2, a_dst2, b2):
    edge_index = edge_index.astype(jnp.int32)
    src = edge_index[0]
    dst = edge_index[1]

    eye8 = jnp.eye(HEADS1, dtype=jnp.float32)
    ms1 = (eye8[:, None, :] * a_src1[:, :, None]).reshape(MSG1, HEADS1)
    md1 = (eye8[:, None, :] * a_dst1[:, :, None]).reshape(MSG1, HEADS1)
    z8 = jnp.zeros((x.shape[1], HEADS1), jnp.float32)
    wbig1 = jnp.concatenate([W1, W1 @ ms1, z8, W1 @ md1, z8], axis=1)

    z15 = jnp.zeros((MSG1, 15), jnp.float32)
    wbig2 = jnp.concatenate(
        [W2, (W2 @ a_src2[0])[:, None], z15, (W2 @ a_dst2[0])[:, None], z15],
        axis=1)
    rmat = jnp.repeat(eye8, 16, axis=1)

    xp = jnp.pad(x, ((0, NP - N), (0, 0)))

    srctab1, dsttab1 = _tables1(xp, wbig1)
    z1 = jnp.zeros((K, ROW1), jnp.float32)
    accp1 = _edge_pass(ROW1, MSG1, HEADS1, srctab1, dsttab1, src, dst, z1)

    srctab2, dsttab2 = _finalize1_matmul2(accp1[0], accp1[1], rmat,
                                          b1.reshape(1, MSG1), wbig2)
    z2 = jnp.zeros((K, ROW2), jnp.float32)
    accp2 = _edge_pass(ROW2, MSG2, 1, srctab2, dsttab2, src, dst, z2)

    out = _finalize2(accp2[0], accp2[1], b2.reshape(1, MSG2))
    return out[:N]
